# baseline (speedup 1.0000x reference)
import functools

import jax
import jax.numpy as jnp
from jax import lax
from jax.experimental import pallas as pl
from jax.experimental.pallas import tpu as pltpu
from jax.experimental.pallas import tpu_sc as plsc

N = 10000
E = 320000
NC = 2
NS = 16
L = 16
CH = 80
NPAD = 10240
BLK = 1000

_MESH = plsc.VectorSubcoreMesh(core_axis_name="c", subcore_axis_name="s")


CH2 = 125
RPT = E // (NS * CH2)
GR = 16
NG = RPT // GR


def _scat_body(yflat, srcoff, dst2d, zrows, out, srcst, dstst, rows2, sst, sg0, sg1, ssc, acc):
    c = lax.axis_index("c")
    s = lax.axis_index("s")
    for t in range(8):
        pltpu.sync_copy(zrows, acc.at[pl.ds(s * 640 + t * CH, CH)])

    tbase = s * RPT
    pltpu.sync_copy(srcoff.at[c, pl.ds(tbase, GR)], srcst.at[0])
    pltpu.sync_copy(dst2d.at[pl.ds(tbase, GR)], dstst.at[0])
    plsc.subcore_barrier()

    sg = (sg0, sg1)
    pltpu.async_copy(yflat.at[srcst.at[0, 0]], rows2.at[0], sg0)

    def body(g, carry):
        gb = lax.rem(g, 2)
        nb2 = lax.rem(g + 1, 2)
        nxt = tbase + (g + 1) * GR

        @pl.when(g < NG - 1)
        def _():
            pltpu.async_copy(srcoff.at[c, pl.ds(nxt, GR)], srcst.at[nb2], sst)
            pltpu.async_copy(dst2d.at[pl.ds(nxt, GR)], dstst.at[nb2], sst)

        for k in range(GR):
            b = k % 2
            nbb = 1 - b
            if k > 0:
                pltpu.make_async_copy(rows2.at[nbb], acc.at[dstst.at[gb, k]], ssc).wait()
            else:

                @pl.when(g > 0)
                def _():
                    pltpu.make_async_copy(
                        rows2.at[nbb], acc.at[dstst.at[gb, k]], ssc
                    ).wait()

            if k < GR - 1:
                pltpu.async_copy(yflat.at[srcst.at[gb, k + 1]], rows2.at[nbb], sg[nbb])
            else:

                @pl.when(g < NG - 1)
                def _():
                    pltpu.make_async_copy(
                        srcoff.at[c, pl.ds(nxt, GR)], srcst.at[nb2], sst
                    ).wait()
                    pltpu.make_async_copy(
                        dst2d.at[pl.ds(nxt, GR)], dstst.at[nb2], sst
                    ).wait()
                    pltpu.async_copy(yflat.at[srcst.at[nb2, 0]], rows2.at[nbb], sg[nbb])

            pltpu.make_async_copy(yflat.at[srcst.at[gb, k]], rows2.at[b], sg[b]).wait()
            pltpu.async_copy(rows2.at[b], acc.at[dstst.at[gb, k]], ssc, add=True)
        return carry

    lax.fori_loop(0, NG, body, 0)
    pltpu.make_async_copy(rows2.at[1], acc.at[dstst.at[0, 0]], ssc).wait()
    plsc.subcore_barrier()
    r0 = s * 640
    pltpu.sync_copy(acc.at[pl.ds(r0, 640)], out.at[c, pl.ds(r0, 640)])


_scatter_rows = pl.kernel(
    _scat_body,
    out_type=jax.ShapeDtypeStruct((NC, NPAD, 128), jnp.float32),
    mesh=_MESH,
    scratch_types=[
        pltpu.VMEM((2, GR, CH2), jnp.int32),
        pltpu.VMEM((2, GR, CH2), jnp.int32),
        pltpu.VMEM((2, CH2, 128), jnp.float32),
        pltpu.SemaphoreType.DMA,
        pltpu.SemaphoreType.DMA,
        pltpu.SemaphoreType.DMA,
        pltpu.SemaphoreType.DMA,
        pltpu.VMEM_SHARED((NPAD, 128), jnp.float32),
    ],
)


DW = 128


def _deg_body(dst2d, zrows, ones_hbm, out, dstst, ones_v, acc):
    c = lax.axis_index("c")
    s = lax.axis_index("s")
    pltpu.sync_copy(zrows, acc.at[pl.ds(s * 640, 640)])
    pltpu.sync_copy(ones_hbm, ones_v)
    plsc.subcore_barrier()
    ngroups = (NG + 1) // 2 - c * (NG % 2)

    def body(g, carry):
        base = s * RPT + (2 * g + c) * GR
        pltpu.sync_copy(dst2d.at[pl.ds(base, GR)], dstst)
        for k in range(GR):
            pltpu.sync_copy(ones_v, acc.at[dstst.at[k]], add=True)
        return carry

    lax.fori_loop(0, ngroups, body, 0)
    plsc.subcore_barrier()
    pltpu.sync_copy(acc.at[pl.ds(s * 640, 640)], out.at[c, pl.ds(s * 640, 640)])


_deg_count = pl.kernel(
    _deg_body,
    out_type=jax.ShapeDtypeStruct((NC, NPAD, DW), jnp.float32),
    mesh=_MESH,
    scratch_types=[
        pltpu.VMEM((GR, CH2), jnp.int32),
        pltpu.VMEM((CH2, DW), jnp.float32),
        pltpu.VMEM_SHARED((NPAD, DW), jnp.float32),
    ],
)


def _mm1_body(degp_ref, x_ref, w1_ref, y_ref, dinv_ref):
    deg = degp_ref[0, :, 0] + degp_ref[1, :, 0] + 1.0
    dinv = lax.rsqrt(deg)
    xw = jnp.dot(x_ref[...], w1_ref[...], preferred_element_type=jnp.float32)
    y = xw * dinv[:, None]
    y_ref[0] = y[:, :128]
    y_ref[1] = y[:, 128:]
    dinv_ref[...] = dinv[:, None]


def _mm1(degp, x, w1):
    return pl.pallas_call(
        _mm1_body,
        grid=(N // BLK,),
        in_specs=[
            pl.BlockSpec((NC, BLK, DW), lambda i: (0, i, 0)),
            pl.BlockSpec((BLK, 128), lambda i: (i, 0)),
            pl.BlockSpec((128, 256), lambda i: (0, 0)),
        ],
        out_specs=[
            pl.BlockSpec((NC, BLK, 128), lambda i: (0, i, 0)),
            pl.BlockSpec((BLK, 1), lambda i: (i, 0)),
        ],
        out_shape=[
            jax.ShapeDtypeStruct((NC, N, 128), jnp.float32),
            jax.ShapeDtypeStruct((N, 1), jnp.float32),
        ],
    )(degp, x, w1)


def _cat(ref):
    return jnp.concatenate([ref[0], ref[1]], axis=-1)


def _mm_mid_body(dinv_ref, s_ref, y_ref, b1_ref, wc_ref, bc_ref, w2_ref, o_ref):
    dinv = dinv_ref[...]
    st = _cat(s_ref) + _cat(y_ref)
    h = jnp.maximum(st * dinv + b1_ref[0][None, :], 0.0)
    hc = jnp.dot(h, wc_ref[...], preferred_element_type=jnp.float32) + bc_ref[0][None, :]
    y2 = jnp.dot(hc, w2_ref[...], preferred_element_type=jnp.float32) * dinv
    o_ref[0] = y2[:, :128]
    o_ref[1] = y2[:, 128:]


def _mm_mid(dinv, s1, y1, b1, wc, bc, w2):
    return pl.pallas_call(
        _mm_mid_body,
        grid=(N // BLK,),
        in_specs=[
            pl.BlockSpec((BLK, 1), lambda i: (i, 0)),
            pl.BlockSpec((NC, BLK, 128), lambda i: (0, i, 0)),
            pl.BlockSpec((NC, BLK, 128), lambda i: (0, i, 0)),
            pl.BlockSpec((1, 256), lambda i: (0, 0)),
            pl.BlockSpec((256, 256), lambda i: (0, 0)),
            pl.BlockSpec((1, 256), lambda i: (0, 0)),
            pl.BlockSpec((256, 256), lambda i: (0, 0)),
        ],
        out_specs=pl.BlockSpec((NC, BLK, 128), lambda i: (0, i, 0)),
        out_shape=jax.ShapeDtypeStruct((NC, N, 128), jnp.float32),
    )(dinv, s1, y1, b1, wc, bc, w2)


def _mm3_body(dinv_ref, s_ref, y_ref, b2_ref, w3_ref, o_ref):
    dinv = dinv_ref[...]
    st = _cat(s_ref) + _cat(y_ref)
    h = jnp.maximum(st * dinv + b2_ref[0][None, :], 0.0)
    y3 = jnp.dot(h, w3_ref[...], preferred_element_type=jnp.float32) * dinv
    o_ref[0] = y3[:, :128]
    o_ref[1] = y3[:, 128:]


def _mm3(dinv, s2, y2, b2, w3):
    return pl.pallas_call(
        _mm3_body,
        grid=(N // BLK,),
        in_specs=[
            pl.BlockSpec((BLK, 1), lambda i: (i, 0)),
            pl.BlockSpec((NC, BLK, 128), lambda i: (0, i, 0)),
            pl.BlockSpec((NC, BLK, 128), lambda i: (0, i, 0)),
            pl.BlockSpec((1, 256), lambda i: (0, 0)),
            pl.BlockSpec((256, 256), lambda i: (0, 0)),
        ],
        out_specs=pl.BlockSpec((NC, BLK, 128), lambda i: (0, i, 0)),
        out_shape=jax.ShapeDtypeStruct((NC, N, 128), jnp.float32),
    )(dinv, s2, y2, b2, w3)


def _final_body(dinv_ref, s_ref, y_ref, b3_ref, wcls_ref, bcls_ref, o_ref, mx_ref):
    i = pl.program_id(0)
    dinv = dinv_ref[...]
    st = _cat(s_ref) + _cat(y_ref)
    h3 = st * dinv + b3_ref[0][None, :]
    bm = jnp.max(h3, axis=0, keepdims=True)

    @pl.when(i == 0)
    def _():
        mx_ref[...] = bm

    @pl.when(i > 0)
    def _():
        mx_ref[...] = jnp.maximum(mx_ref[...], bm)

    @pl.when(i == N // BLK - 1)
    def _():
        pooled = mx_ref[...]
        logits = (
            jnp.dot(pooled, wcls_ref[...], preferred_element_type=jnp.float32)
            + bcls_ref[0][None, :]
        )
        m = jnp.max(logits, axis=1, keepdims=True)
        e = jnp.exp(logits - m)
        o_ref[...] = e / jnp.sum(e, axis=1, keepdims=True)


def _final(dinv, s3, y3, b3, wcls, bcls):
    return pl.pallas_call(
        _final_body,
        grid=(N // BLK,),
        in_specs=[
            pl.BlockSpec((BLK, 1), lambda i: (i, 0)),
            pl.BlockSpec((NC, BLK, 128), lambda i: (0, i, 0)),
            pl.BlockSpec((NC, BLK, 128), lambda i: (0, i, 0)),
            pl.BlockSpec((1, 256), lambda i: (0, 0)),
            pl.BlockSpec((256, 2), lambda i: (0, 0)),
            pl.BlockSpec((1, 2), lambda i: (0, 0)),
        ],
        out_specs=pl.BlockSpec((1, 2), lambda i: (0, 0)),
        out_shape=jax.ShapeDtypeStruct((1, 2), jnp.float32),
        scratch_shapes=[pltpu.VMEM((1, 256), jnp.float32)],
    )(dinv, s3, y3, b3, wcls, bcls)


def kernel(x, edge_index, W1, b1, Wc, bc, W2, b2, W3, b3, Wcls, bcls):
    ei = edge_index.astype(jnp.int32)
    src = ei[0]
    dst = ei[1]
    srcoff = jnp.stack([src, src + N]).reshape(NC, E // CH2, CH2)
    dst2d = dst.reshape(E // CH2, CH2)
    z128 = jnp.zeros((CH, 128), jnp.float32)
    zdeg = jnp.zeros((640, DW), jnp.float32)
    onesdeg = jnp.ones((CH2, DW), jnp.float32)

    degp = _deg_count(dst2d, zdeg, onesdeg)
    y1, dinv = _mm1(degp, x, W1)
    s1 = _scatter_rows(y1.reshape(NC * N, 128), srcoff, dst2d, z128)
    y2 = _mm_mid(dinv, s1, y1, b1[None, :], Wc, bc[None, :], W2)
    s2 = _scatter_rows(y2.reshape(NC * N, 128), srcoff, dst2d, z128)
    y3 = _mm3(dinv, s2, y2, b2[None, :], W3)
    s3 = _scatter_rows(y3.reshape(NC * N, 128), srcoff, dst2d, z128)
    res = _final(dinv, s3, y3, b3[None, :], Wcls, bcls[None, :])
    return (res, x)


# speedup vs baseline: 1.0040x; 1.0040x over previous
import jax
import jax.numpy as jnp
from jax import lax
from jax.experimental import pallas as pl
from jax.experimental.pallas import tpu as pltpu
from jax.experimental.pallas import tpu_sc as plsc

N = 10000
E = 320000
NC = 2
NS = 16
L = 16
CH = 80
NPAD = 10240
BLK = 1000

_MESH = plsc.VectorSubcoreMesh(core_axis_name="c", subcore_axis_name="s")


CH2 = 125
RPT = E // (NS * CH2)
GR = 8
NG = RPT // GR


def _scat_body(yflat, srcoff, dst2d, zrows, out, srcst, dstst, rows2, sst, sg0, sg1, ssc, acc):
    c = lax.axis_index("c")
    s = lax.axis_index("s")
    for t in range(8):
        pltpu.sync_copy(zrows, acc.at[pl.ds(s * 640 + t * CH, CH)])

    tbase = s * RPT
    pltpu.sync_copy(srcoff.at[c, pl.ds(tbase, GR)], srcst.at[0])
    pltpu.sync_copy(dst2d.at[pl.ds(tbase, GR)], dstst.at[0])
    plsc.subcore_barrier()

    sg = (sg0, sg1)
    pltpu.async_copy(yflat.at[srcst.at[0, 0]], rows2.at[0], sg0)

    def body(g, carry):
        gb = lax.rem(g, 2)
        nb2 = lax.rem(g + 1, 2)
        nxt = tbase + (g + 1) * GR

        @pl.when(g < NG - 1)
        def _():
            pltpu.async_copy(srcoff.at[c, pl.ds(nxt, GR)], srcst.at[nb2], sst)
            pltpu.async_copy(dst2d.at[pl.ds(nxt, GR)], dstst.at[nb2], sst)

        for k in range(GR):
            b = k % 2
            nbb = 1 - b
            if k > 0:
                pltpu.make_async_copy(rows2.at[nbb], acc.at[dstst.at[gb, k]], ssc).wait()
            else:

                @pl.when(g > 0)
                def _():
                    pltpu.make_async_copy(
                        rows2.at[nbb], acc.at[dstst.at[gb, k]], ssc
                    ).wait()

            if k < GR - 1:
                pltpu.async_copy(yflat.at[srcst.at[gb, k + 1]], rows2.at[nbb], sg[nbb])
            else:

                @pl.when(g < NG - 1)
                def _():
                    pltpu.make_async_copy(
                        srcoff.at[c, pl.ds(nxt, GR)], srcst.at[nb2], sst
                    ).wait()
                    pltpu.make_async_copy(
                        dst2d.at[pl.ds(nxt, GR)], dstst.at[nb2], sst
                    ).wait()
                    pltpu.async_copy(yflat.at[srcst.at[nb2, 0]], rows2.at[nbb], sg[nbb])

            pltpu.make_async_copy(yflat.at[srcst.at[gb, k]], rows2.at[b], sg[b]).wait()
            pltpu.async_copy(rows2.at[b], acc.at[dstst.at[gb, k]], ssc, add=True)
        return carry

    lax.fori_loop(0, NG, body, 0)
    pltpu.make_async_copy(rows2.at[1], acc.at[dstst.at[0, 0]], ssc).wait()
    plsc.subcore_barrier()
    r0 = s * 640
    pltpu.sync_copy(acc.at[pl.ds(r0, 640)], out.at[c, pl.ds(r0, 640)])


_scatter_rows = pl.kernel(
    _scat_body,
    out_type=jax.ShapeDtypeStruct((NC, NPAD, 128), jnp.float32),
    mesh=_MESH,
    scratch_types=[
        pltpu.VMEM((2, GR, CH2), jnp.int32),
        pltpu.VMEM((2, GR, CH2), jnp.int32),
        pltpu.VMEM((2, CH2, 128), jnp.float32),
        pltpu.SemaphoreType.DMA,
        pltpu.SemaphoreType.DMA,
        pltpu.SemaphoreType.DMA,
        pltpu.SemaphoreType.DMA,
        pltpu.VMEM_SHARED((NPAD, 128), jnp.float32),
    ],
)


DW = 128


def _deg_body(dst2d, zrows, ones_hbm, out, dstst, ones_v, acc):
    c = lax.axis_index("c")
    s = lax.axis_index("s")
    pltpu.sync_copy(zrows, acc.at[pl.ds(s * 640, 640)])
    pltpu.sync_copy(ones_hbm, ones_v)
    plsc.subcore_barrier()
    ngroups = (NG + 1) // 2 - c * (NG % 2)

    def body(g, carry):
        base = s * RPT + (2 * g + c) * GR
        pltpu.sync_copy(dst2d.at[pl.ds(base, GR)], dstst)
        for k in range(GR):
            pltpu.sync_copy(ones_v, acc.at[dstst.at[k]], add=True)
        return carry

    lax.fori_loop(0, ngroups, body, 0)
    plsc.subcore_barrier()
    pltpu.sync_copy(acc.at[pl.ds(s * 640, 640)], out.at[c, pl.ds(s * 640, 640)])


_deg_count = pl.kernel(
    _deg_body,
    out_type=jax.ShapeDtypeStruct((NC, NPAD, DW), jnp.float32),
    mesh=_MESH,
    scratch_types=[
        pltpu.VMEM((GR, CH2), jnp.int32),
        pltpu.VMEM((CH2, DW), jnp.float32),
        pltpu.VMEM_SHARED((NPAD, DW), jnp.float32),
    ],
)


def _mm1_body(degp_ref, x_ref, w1_ref, y_ref, dinv_ref):
    deg = degp_ref[0, :, 0] + degp_ref[1, :, 0] + 1.0
    dinv = lax.rsqrt(deg)
    xw = jnp.dot(x_ref[...], w1_ref[...], preferred_element_type=jnp.float32)
    y = xw * dinv[:, None]
    y_ref[0] = y[:, :128]
    y_ref[1] = y[:, 128:]
    dinv_ref[...] = dinv[:, None]


def _mm1(degp, x, w1):
    return pl.pallas_call(
        _mm1_body,
        grid=(N // BLK,),
        in_specs=[
            pl.BlockSpec((NC, BLK, DW), lambda i: (0, i, 0)),
            pl.BlockSpec((BLK, 128), lambda i: (i, 0)),
            pl.BlockSpec((128, 256), lambda i: (0, 0)),
        ],
        out_specs=[
            pl.BlockSpec((NC, BLK, 128), lambda i: (0, i, 0)),
            pl.BlockSpec((BLK, 1), lambda i: (i, 0)),
        ],
        out_shape=[
            jax.ShapeDtypeStruct((NC, N, 128), jnp.float32),
            jax.ShapeDtypeStruct((N, 1), jnp.float32),
        ],
    )(degp, x, w1)


def _cat(ref):
    return jnp.concatenate([ref[0], ref[1]], axis=-1)


def _mm_mid_body(dinv_ref, s_ref, y_ref, b1_ref, wc_ref, bc_ref, w2_ref, o_ref):
    dinv = dinv_ref[...]
    st = _cat(s_ref) + _cat(y_ref)
    h = jnp.maximum(st * dinv + b1_ref[0][None, :], 0.0)
    hc = jnp.dot(h, wc_ref[...], preferred_element_type=jnp.float32) + bc_ref[0][None, :]
    y2 = jnp.dot(hc, w2_ref[...], preferred_element_type=jnp.float32) * dinv
    o_ref[0] = y2[:, :128]
    o_ref[1] = y2[:, 128:]


def _mm_mid(dinv, s1, y1, b1, wc, bc, w2):
    return pl.pallas_call(
        _mm_mid_body,
        grid=(N // BLK,),
        in_specs=[
            pl.BlockSpec((BLK, 1), lambda i: (i, 0)),
            pl.BlockSpec((NC, BLK, 128), lambda i: (0, i, 0)),
            pl.BlockSpec((NC, BLK, 128), lambda i: (0, i, 0)),
            pl.BlockSpec((1, 256), lambda i: (0, 0)),
            pl.BlockSpec((256, 256), lambda i: (0, 0)),
            pl.BlockSpec((1, 256), lambda i: (0, 0)),
            pl.BlockSpec((256, 256), lambda i: (0, 0)),
        ],
        out_specs=pl.BlockSpec((NC, BLK, 128), lambda i: (0, i, 0)),
        out_shape=jax.ShapeDtypeStruct((NC, N, 128), jnp.float32),
    )(dinv, s1, y1, b1, wc, bc, w2)


def _mm3_body(dinv_ref, s_ref, y_ref, b2_ref, w3_ref, o_ref):
    dinv = dinv_ref[...]
    st = _cat(s_ref) + _cat(y_ref)
    h = jnp.maximum(st * dinv + b2_ref[0][None, :], 0.0)
    y3 = jnp.dot(h, w3_ref[...], preferred_element_type=jnp.float32) * dinv
    o_ref[0] = y3[:, :128]
    o_ref[1] = y3[:, 128:]


def _mm3(dinv, s2, y2, b2, w3):
    return pl.pallas_call(
        _mm3_body,
        grid=(N // BLK,),
        in_specs=[
            pl.BlockSpec((BLK, 1), lambda i: (i, 0)),
            pl.BlockSpec((NC, BLK, 128), lambda i: (0, i, 0)),
            pl.BlockSpec((NC, BLK, 128), lambda i: (0, i, 0)),
            pl.BlockSpec((1, 256), lambda i: (0, 0)),
            pl.BlockSpec((256, 256), lambda i: (0, 0)),
        ],
        out_specs=pl.BlockSpec((NC, BLK, 128), lambda i: (0, i, 0)),
        out_shape=jax.ShapeDtypeStruct((NC, N, 128), jnp.float32),
    )(dinv, s2, y2, b2, w3)


def _final_body(dinv_ref, s_ref, y_ref, b3_ref, wcls_ref, bcls_ref, o_ref, mx_ref):
    i = pl.program_id(0)
    dinv = dinv_ref[...]
    st = _cat(s_ref) + _cat(y_ref)
    h3 = st * dinv + b3_ref[0][None, :]
    bm = jnp.max(h3, axis=0, keepdims=True)

    @pl.when(i == 0)
    def _():
        mx_ref[...] = bm

    @pl.when(i > 0)
    def _():
        mx_ref[...] = jnp.maximum(mx_ref[...], bm)

    @pl.when(i == N // BLK - 1)
    def _():
        pooled = mx_ref[...]
        logits = (
            jnp.dot(pooled, wcls_ref[...], preferred_element_type=jnp.float32)
            + bcls_ref[0][None, :]
        )
        m = jnp.max(logits, axis=1, keepdims=True)
        e = jnp.exp(logits - m)
        o_ref[...] = e / jnp.sum(e, axis=1, keepdims=True)


def _final(dinv, s3, y3, b3, wcls, bcls):
    return pl.pallas_call(
        _final_body,
        grid=(N // BLK,),
        in_specs=[
            pl.BlockSpec((BLK, 1), lambda i: (i, 0)),
            pl.BlockSpec((NC, BLK, 128), lambda i: (0, i, 0)),
            pl.BlockSpec((NC, BLK, 128), lambda i: (0, i, 0)),
            pl.BlockSpec((1, 256), lambda i: (0, 0)),
            pl.BlockSpec((256, 2), lambda i: (0, 0)),
            pl.BlockSpec((1, 2), lambda i: (0, 0)),
        ],
        out_specs=pl.BlockSpec((1, 2), lambda i: (0, 0)),
        out_shape=jax.ShapeDtypeStruct((1, 2), jnp.float32),
        scratch_shapes=[pltpu.VMEM((1, 256), jnp.float32)],
    )(dinv, s3, y3, b3, wcls, bcls)


def kernel(x, edge_index, W1, b1, Wc, bc, W2, b2, W3, b3, Wcls, bcls):
    ei = edge_index.astype(jnp.int32)
    src = ei[0]
    dst = ei[1]
    srcoff = jnp.stack([src, src + N]).reshape(NC, E // CH2, CH2)
    dst2d = dst.reshape(E // CH2, CH2)
    z128 = jnp.zeros((CH, 128), jnp.float32)
    zdeg = jnp.zeros((640, DW), jnp.float32)
    onesdeg = jnp.ones((CH2, DW), jnp.float32)

    degp = _deg_count(dst2d, zdeg, onesdeg)
    y1, dinv = _mm1(degp, x, W1)
    s1 = _scatter_rows(y1.reshape(NC * N, 128), srcoff, dst2d, z128)
    y2 = _mm_mid(dinv, s1, y1, b1[None, :], Wc, bc[None, :], W2)
    s2 = _scatter_rows(y2.reshape(NC * N, 128), srcoff, dst2d, z128)
    y3 = _mm3(dinv, s2, y2, b2[None, :], W3)
    s3 = _scatter_rows(y3.reshape(NC * N, 128), srcoff, dst2d, z128)
    res = _final(dinv, s3, y3, b3[None, :], Wcls, bcls[None, :])
    return (res, x)
